# baseline (speedup 1.0000x reference)
import functools

import jax
import jax.numpy as jnp
from jax import lax
from jax.experimental import pallas as pl
from jax.experimental.pallas import tpu as pltpu
from jax.experimental.pallas import tpu_sc as plsc

NC = 2
NS = 16
L = 16
NW = NC * NS

N = 100000
D = 128
ROWS_PER_W = N // NW
CHUNK = 125
NCHUNKS = ROWS_PER_W // CHUNK
GROUPS = (CHUNK + L - 1) // L
NFC = D // L

_NEG_INF = float("-inf")
_IMAX = jnp.iinfo(jnp.int32).max


def _splat(vec, idx):
    return lax.gather(
        vec, idx[:, None],
        dimension_numbers=lax.GatherDimensionNumbers(
            offset_dims=(), collapsed_slice_dims=(0,), start_index_map=(0,)),
        slice_sizes=(1,),
        mode=lax.GatherScatterMode.PROMISE_IN_BOUNDS)


def _rsqrt(s):
    r = plsc.bitcast(
        jnp.int32(0x5F3759DF) - (plsc.bitcast(s, jnp.int32) >> 1),
        jnp.float32)
    half = 0.5 * s
    for _ in range(4):
        r = r * (1.5 - half * r * r)
    return r


def _merge(bv, bi, v, i):
    upd = (v > bv) | ((v == bv) & (i < bi))
    return jnp.where(upd, v, bv), jnp.where(upd, i, bi)


def _sc_body(keys_hbm, q_hbm, values_hbm,
             rows_out, vals_out, idx_out,
             q_vmem, buf, cand_v, cand_i, merged_v, merged_i,
             stat_v, stat_i, row_vmem,
             shared_v, shared_i, sem0, sem1):
    cid = lax.axis_index("c")
    sid = lax.axis_index("s")
    w = cid * NS + sid
    base = w * ROWS_PER_W

    pltpu.sync_copy(q_hbm, q_vmem)
    q_regs = [q_vmem[pl.ds(fc * L, L)] for fc in range(NFC)]
    row_iota = lax.iota(jnp.int32, L)
    lane_idx = [jnp.full((L,), l, jnp.int32) for l in range(L)]

    def chunk_start(c, slot, sem):
        return pltpu.async_copy(
            keys_hbm.at[pl.ds(base + c * CHUNK, CHUNK)], buf.at[slot], sem)

    def chunk_wait(c, slot, sem):
        pltpu.make_async_copy(
            keys_hbm.at[pl.ds(base + c * CHUNK, CHUNK)], buf.at[slot], sem
        ).wait()

    def process(c, slot, best_v, best_i):
        chunk_base = base + c * CHUNK

        def group(g, carry):
            bv, bi = carry
            row_local = row_iota + g * L
            valid = row_local < CHUNK
            row_cl = jnp.minimum(row_local, CHUNK - 1)
            acc_d = jnp.zeros((L,), jnp.float32)
            acc_s = jnp.zeros((L,), jnp.float32)
            for fc in range(NFC):
                qv = q_regs[fc]
                for l in range(L):
                    f = fc * L + l
                    qf = _splat(qv, lane_idx[l])
                    col = jnp.full((L,), f, jnp.int32)
                    kv = plsc.load_gather(buf.at[slot], [row_cl, col])
                    acc_d = acc_d + kv * qf
                    acc_s = acc_s + kv * kv
            r = _rsqrt(acc_s)
            sim = jnp.where(valid, acc_d * jnp.minimum(r, 1e8), _NEG_INF)
            upd = sim > bv
            return (jnp.where(upd, sim, bv),
                    jnp.where(upd, chunk_base + row_local, bi))

        return lax.fori_loop(0, GROUPS, group, (best_v, best_i))

    best_v = jnp.full((L,), _NEG_INF, jnp.float32)
    best_i = jnp.zeros((L,), jnp.int32)

    chunk_start(0, 0, sem0)

    def pair(cc, carry):
        bv, bi = carry
        c0 = 2 * cc
        chunk_start(c0 + 1, 1, sem1)
        chunk_wait(c0, 0, sem0)
        bv, bi = process(c0, 0, bv, bi)
        chunk_start(c0 + 2, 0, sem0)
        chunk_wait(c0 + 1, 1, sem1)
        bv, bi = process(c0 + 1, 1, bv, bi)
        return bv, bi

    best_v, best_i = lax.fori_loop(0, (NCHUNKS - 1) // 2, pair,
                                   (best_v, best_i))
    chunk_wait(NCHUNKS - 1, 0, sem0)
    best_v, best_i = process(NCHUNKS - 1, 0, best_v, best_i)

    cand_v[...] = best_v
    cand_i[...] = best_i
    pltpu.sync_copy(cand_v, shared_v.at[sid])
    pltpu.sync_copy(cand_i, shared_i.at[sid])
    plsc.subcore_barrier()

    @pl.when(sid == 0)
    def _():
        pltpu.sync_copy(shared_v, merged_v)
        pltpu.sync_copy(shared_i, merged_i)
        bv = merged_v[0, :]
        bi = merged_i[0, :]
        for s in range(1, NS):
            bv, bi = _merge(bv, bi, merged_v[s, :], merged_i[s, :])
        m = jnp.max(bv)
        midx = jnp.min(jnp.where(bv == m, bi, _IMAX))
        pltpu.sync_copy(values_hbm.at[pl.ds(midx, 1)], row_vmem)
        pltpu.sync_copy(row_vmem, rows_out.at[pl.ds(cid, 1)])
        stat_v[...] = jnp.full((L,), m, jnp.float32)
        stat_i[...] = jnp.full((L,), midx, jnp.int32)
        pltpu.sync_copy(stat_v, vals_out.at[cid])
        pltpu.sync_copy(stat_i, idx_out.at[cid])


@jax.jit
def kernel(query, keys, values):
    mesh = plsc.VectorSubcoreMesh(core_axis_name="c", subcore_axis_name="s")
    rows, vals, idxs = pl.kernel(
        _sc_body,
        out_type=(
            jax.ShapeDtypeStruct((NC, D), jnp.float32),
            jax.ShapeDtypeStruct((NC, L), jnp.float32),
            jax.ShapeDtypeStruct((NC, L), jnp.int32),
        ),
        mesh=mesh,
        compiler_params=pltpu.CompilerParams(
            use_tc_tiling_on_sc=False, needs_layout_passes=False),
        scratch_types=[
            pltpu.VMEM((D,), jnp.float32),
            pltpu.VMEM((2, CHUNK, D), jnp.float32),
            pltpu.VMEM((L,), jnp.float32),
            pltpu.VMEM((L,), jnp.int32),
            pltpu.VMEM((NS, L), jnp.float32),
            pltpu.VMEM((NS, L), jnp.int32),
            pltpu.VMEM((L,), jnp.float32),
            pltpu.VMEM((L,), jnp.int32),
            pltpu.VMEM((1, D), jnp.float32),
            pltpu.VMEM_SHARED((NS, L), jnp.float32),
            pltpu.VMEM_SHARED((NS, L), jnp.int32),
            pltpu.SemaphoreType.DMA,
            pltpu.SemaphoreType.DMA,
        ],
    )(keys, query, values)

    v0, v1 = vals[0, 0], vals[1, 0]
    i0, i1 = idxs[0, 0], idxs[1, 0]
    pick0 = (v0 > v1) | ((v0 == v1) & (i0 <= i1))
    return jnp.where(pick0, rows[0], rows[1])


# speedup vs baseline: 4.4431x; 4.4431x over previous
import functools

import jax
import jax.numpy as jnp
from jax import lax
from jax.experimental import pallas as pl
from jax.experimental.pallas import tpu as pltpu
from jax.experimental.pallas import tpu_sc as plsc

NC = 2
NS = 16
L = 16
NW = NC * NS

N = 100000
D = 128
ROWS_PER_W = N // NW
CHUNK = 125
NCHUNKS = ROWS_PER_W // CHUNK
GROUPS = (CHUNK + L - 1) // L
NFC = D // L

_NEG_INF = float("-inf")
_S_MIN = 1e-30


def _tree_sum(vs):
    while len(vs) > 1:
        vs = [a + b for a, b in zip(vs[::2], vs[1::2])]
    return vs[0]


def _sc_body(keys_hbm, q_hbm, values_hbm,
             rows_out, num_out, s_out, idx_out,
             q_vmem, buf, cand_n, cand_s, cand_i,
             merged_n, merged_s, merged_i,
             stat_vec, row_vmem,
             shared_n, shared_s, shared_i, sem0, sem1):
    cid = lax.axis_index("c")
    sid = lax.axis_index("s")
    w = cid * NS + sid
    base = w * ROWS_PER_W

    pltpu.sync_copy(q_hbm, q_vmem)
    q_regs = [q_vmem[pl.ds(fc * L, L)] for fc in range(NFC)]

    def chunk_src(c):
        return keys_hbm.at[pl.ds(base + c * CHUNK, CHUNK)]

    def process(c, slot, carry):
        chunk_base = base + c * CHUNK

        def group(g, carry):
            bn, bs, bi = carry
            gbase = g * L
            for r in range(L):
                row = jnp.minimum(gbase + r, CHUNK - 1)
                kvs = [buf[slot, row, pl.ds(fc * L, L)] for fc in range(NFC)]
                d = jnp.sum(_tree_sum([kv * qv for kv, qv in zip(kvs, q_regs)]))
                s = jnp.sum(_tree_sum([kv * kv for kv in kvs]))
                s = jnp.maximum(s, _S_MIN)
                n = d * jnp.abs(d)
                upd = (n * bs > bn * s) & (gbase + r < CHUNK)
                bn = jnp.where(upd, n, bn)
                bs = jnp.where(upd, s, bs)
                bi = jnp.where(upd, chunk_base + gbase + r, bi)
            return bn, bs, bi

        return lax.fori_loop(0, GROUPS, group, carry)

    carry = (jnp.float32(_NEG_INF), jnp.float32(1.0), jnp.int32(0))

    pltpu.async_copy(chunk_src(0), buf.at[0], sem0)

    def step(c, carry):
        slot = lax.rem(c, 2)
        nslot = 1 - slot
        sem_cur = [sem0, sem1]

        @pl.when(c + 1 < NCHUNKS)
        def _():
            @pl.when(nslot == 1)
            def _():
                pltpu.async_copy(chunk_src(c + 1), buf.at[1], sem1)

            @pl.when(nslot == 0)
            def _():
                pltpu.async_copy(chunk_src(c + 1), buf.at[0], sem0)

        @pl.when(slot == 0)
        def _():
            pltpu.make_async_copy(chunk_src(c), buf.at[0], sem0).wait()

        @pl.when(slot == 1)
        def _():
            pltpu.make_async_copy(chunk_src(c), buf.at[1], sem1).wait()

        return process(c, slot, carry)

    bn, bs, bi = lax.fori_loop(0, NCHUNKS, step, carry)

    cand_n[...] = jnp.full((L,), bn, jnp.float32)
    cand_s[...] = jnp.full((L,), bs, jnp.float32)
    cand_i[...] = jnp.full((L,), bi, jnp.int32)
    pltpu.sync_copy(cand_n, shared_n.at[sid])
    pltpu.sync_copy(cand_s, shared_s.at[sid])
    pltpu.sync_copy(cand_i, shared_i.at[sid])
    plsc.subcore_barrier()

    @pl.when(sid == 0)
    def _():
        pltpu.sync_copy(shared_n, merged_n)
        pltpu.sync_copy(shared_s, merged_s)
        pltpu.sync_copy(shared_i, merged_i)
        bn = merged_n[0, :]
        bs = merged_s[0, :]
        bi = merged_i[0, :]
        for t in range(1, NS):
            n = merged_n[t, :]
            s = merged_s[t, :]
            i = merged_i[t, :]
            a = n * bs
            b = bn * s
            upd = (a > b) | ((a == b) & (i < bi))
            bn = jnp.where(upd, n, bn)
            bs = jnp.where(upd, s, bs)
            bi = jnp.where(upd, i, bi)
        midx = jnp.max(bi)
        pltpu.sync_copy(values_hbm.at[pl.ds(midx, 1)], row_vmem)
        pltpu.sync_copy(row_vmem, rows_out.at[pl.ds(cid, 1)])
        stat_vec[...] = bn
        pltpu.sync_copy(stat_vec, num_out.at[cid])
        stat_vec[...] = bs
        pltpu.sync_copy(stat_vec, s_out.at[cid])
        cand_i[...] = bi
        pltpu.sync_copy(cand_i, idx_out.at[cid])


@jax.jit
def kernel(query, keys, values):
    mesh = plsc.VectorSubcoreMesh(core_axis_name="c", subcore_axis_name="s")
    rows, nums, ss, idxs = pl.kernel(
        _sc_body,
        out_type=(
            jax.ShapeDtypeStruct((NC, D), jnp.float32),
            jax.ShapeDtypeStruct((NC, L), jnp.float32),
            jax.ShapeDtypeStruct((NC, L), jnp.float32),
            jax.ShapeDtypeStruct((NC, L), jnp.int32),
        ),
        mesh=mesh,
        compiler_params=pltpu.CompilerParams(
            use_tc_tiling_on_sc=False, needs_layout_passes=False),
        scratch_types=[
            pltpu.VMEM((D,), jnp.float32),
            pltpu.VMEM((2, CHUNK, D), jnp.float32),
            pltpu.VMEM((L,), jnp.float32),
            pltpu.VMEM((L,), jnp.float32),
            pltpu.VMEM((L,), jnp.int32),
            pltpu.VMEM((NS, L), jnp.float32),
            pltpu.VMEM((NS, L), jnp.float32),
            pltpu.VMEM((NS, L), jnp.int32),
            pltpu.VMEM((L,), jnp.float32),
            pltpu.VMEM((1, D), jnp.float32),
            pltpu.VMEM_SHARED((NS, L), jnp.float32),
            pltpu.VMEM_SHARED((NS, L), jnp.float32),
            pltpu.VMEM_SHARED((NS, L), jnp.int32),
            pltpu.SemaphoreType.DMA,
            pltpu.SemaphoreType.DMA,
        ],
    )(keys, query, values)

    n0, n1 = nums[0, 0], nums[1, 0]
    s0, s1 = ss[0, 0], ss[1, 0]
    i0, i1 = idxs[0, 0], idxs[1, 0]
    a, b = n0 * s1, n1 * s0
    pick0 = (a > b) | ((a == b) & (i0 <= i1))
    return jnp.where(pick0, rows[0], rows[1])
